# baseline (speedup 1.0000x reference)
import functools

import jax
import jax.numpy as jnp
from jax import lax
from jax.experimental import pallas as pl
from jax.experimental.pallas import tpu as pltpu
from jax.experimental.pallas import tpu_sc as plsc

B = 16384
D = 128
RD = 128
NREL = 64
LAM = 1e-05

NC = 2
NS = 16
NW = NC * NS

STILE = 128
NT_MAX = B // STILE + NREL
NSLOT = NT_MAX * STILE
SAMP_PER_W = B // NW
SLOTS_PER_W = NSLOT // NW
CHUNK = 128


def _wid():
    return lax.axis_index("s") * NC + lax.axis_index("c")


def _mesh():
    return plsc.VectorSubcoreMesh(core_axis_name="c", subcore_axis_name="s")


def _last(v):
    return lax.squeeze(lax.slice(v, (15,), (16,)), dimensions=(0,))


def _first(v):
    return lax.squeeze(lax.slice(v, (0,), (1,)), dimensions=(0,))


def _lanes():
    return lax.broadcasted_iota(jnp.int32, (16,), 0)


def _take(v, idx):
    dnums = lax.GatherDimensionNumbers(
        offset_dims=(), collapsed_slice_dims=(0,), start_index_map=(0,))
    return lax.gather(v, idx[:, None], dnums, (1,),
                      mode=lax.GatherScatterMode.PROMISE_IN_BOUNDS)


def _vsum_splat(v):
    lanes = _lanes()
    for k in (1, 2, 4, 8):
        v = v + _take(v, lanes ^ k)
    return v


def _vcumsum(v):
    lanes = _lanes()
    for k in (1, 2, 4, 8):
        sh = _take(v, jnp.maximum(lanes - k, 0))
        v = v + jnp.where(lanes >= k, sh, 0)
    return v


def _vcummax(v):
    lanes = _lanes()
    for k in (1, 2, 4, 8):
        sh = _take(v, jnp.maximum(lanes - k, 0))
        v = jnp.maximum(v, jnp.where(lanes >= k, sh, v))
    return v


def _hist_body(r_hbm, lh_hbm, rs_v, hist_v):
    wid = _wid()
    pltpu.sync_copy(r_hbm.at[pl.ds(wid * SAMP_PER_W, SAMP_PER_W)], rs_v)
    lanes = _lanes()
    one = jnp.zeros((16,), jnp.int32) + 1
    z = jnp.zeros((16,), jnp.int32)
    for g in range(4):
        hist_v[pl.ds(g * 16, 16)] = z

    def cbody(i, c):
        bins = rs_v[pl.ds(i * 16, 16)]
        for g in range(4):
            upd = jnp.zeros((16,), jnp.int32)
            for ii in range(16):
                bi = _take(bins, lanes * 0 + ii)
                upd = upd + jnp.where(bi == g * 16 + lanes, one, 0)
            hist_v[pl.ds(g * 16, 16)] = hist_v[pl.ds(g * 16, 16)] + upd
        return c

    lax.fori_loop(0, SAMP_PER_W // 16, cbody, jnp.int32(0))
    pltpu.sync_copy(hist_v, lh_hbm.at[pl.ds(wid * NREL, NREL)])


def _hist(r):
    f = pl.kernel(
        _hist_body,
        out_type=jax.ShapeDtypeStruct((NW * NREL,), jnp.int32),
        mesh=_mesh(),
        scratch_types=[
            pltpu.VMEM((SAMP_PER_W,), jnp.int32),
            pltpu.VMEM((NREL,), jnp.int32),
        ],
    )
    return f(r)


def _route_body(r_hbm, lh_hbm, h_hbm, p_hbm, n_hbm,
                hs_hbm, ps_hbm, ns_hbm, rel_hbm, valid_hbm,
                rs_v, lh_v, hv_v, pv_v, nv_v,
                woff_v, pos2_v, tl_v, vd_v, sem):
    wid = _wid()
    pltpu.sync_copy(lh_hbm, lh_v)
    pltpu.sync_copy(r_hbm.at[pl.ds(wid * SAMP_PER_W, SAMP_PER_W)], rs_v)
    pltpu.sync_copy(h_hbm.at[pl.ds(wid * SAMP_PER_W, SAMP_PER_W)], hv_v)
    pltpu.sync_copy(p_hbm.at[pl.ds(wid * SAMP_PER_W, SAMP_PER_W)], pv_v)
    pltpu.sync_copy(n_hbm.at[pl.ds(wid * SAMP_PER_W, SAMP_PER_W)], nv_v)
    lanes = _lanes()
    one = jnp.zeros((16,), jnp.int32) + 1
    widv = jnp.zeros((16,), jnp.int32) + wid

    cvecs, pvecs = [], []
    for g in range(4):
        cg = jnp.zeros((16,), jnp.int32)
        pg = jnp.zeros((16,), jnp.int32)
        for w2 in range(NW):
            row = lh_v[pl.ds(w2 * NREL + g * 16, 16)]
            cg = cg + row
            pg = pg + row * jnp.clip(widv - w2, 0, 1)
        cvecs.append(cg)
        pvecs.append(pg)
    tvecs = [lax.shift_right_logical(c + (STILE - 1), 7) for c in cvecs]
    tsvecs = []
    carry = jnp.int32(0)
    for g in range(4):
        incl = _vcumsum(tvecs[g])
        tsvecs.append(incl + carry - tvecs[g])
        carry = carry + _last(incl)
    for g in range(4):
        woff_v[pl.ds(g * 16, 16)] = tsvecs[g] * STILE + pvecs[g]

    def cbody(i, c):
        bins = rs_v[pl.ds(i * 16, 16)]
        g4 = lax.shift_right_logical(bins, 4)
        l4 = bins & 15
        base = jnp.zeros((16,), jnp.int32)
        for g in range(4):
            wg = woff_v[pl.ds(g * 16, 16)]
            base = base + jnp.where(g4 == g, _take(wg, l4), 0)
        rank = jnp.zeros((16,), jnp.int32)
        for k in range(1, 16):
            sh = _take(bins, jnp.maximum(lanes - k, 0))
            rank = rank + jnp.where(lanes >= k,
                                    jnp.where(sh == bins, one, 0), 0)
        pos2_v[lax.shift_right_logical(i, 3),
               pl.ds((i & 7) * 16, 16)] = base + rank
        for g in range(4):
            upd = jnp.zeros((16,), jnp.int32)
            for ii in range(16):
                bi = _take(bins, lanes * 0 + ii)
                upd = upd + jnp.where(bi == g * 16 + lanes, one, 0)
            woff_v[pl.ds(g * 16, 16)] = woff_v[pl.ds(g * 16, 16)] + upd
        return c

    lax.fori_loop(0, SAMP_PER_W // 16, cbody, jnp.int32(0))

    handles = [
        pltpu.async_copy(data_v.at[pl.ds(c * CHUNK, CHUNK)],
                         out_hbm.at[pos2_v.at[c]], sem)
        for data_v, out_hbm in ((hv_v, hs_hbm), (pv_v, ps_hbm),
                                (nv_v, ns_hbm))
        for c in range(SAMP_PER_W // CHUNK)
    ]
    for hdl in handles:
        hdl.wait()

    @pl.when(wid == 0)
    def _meta():
        def mbody(i, c):
            tt = lanes + i * 16
            cnt = jnp.zeros((16,), jnp.int32)
            for g in range(4):
                for k in range(16):
                    tsjk = _take(tsvecs[g], lanes * 0 + k)
                    cnt = cnt + jnp.where(tsjk <= tt, one, 0)
            rr = cnt - 1
            tl_v[pl.ds(i * 16, 16)] = rr
            c_sel = jnp.zeros((16,), jnp.int32)
            ts_sel = jnp.zeros((16,), jnp.int32)
            for g in range(4):
                gm = lax.shift_right_logical(rr, 4) == g
                c_sel = c_sel + jnp.where(gm, _take(cvecs[g], rr & 15), 0)
                ts_sel = ts_sel + jnp.where(gm, _take(tsvecs[g], rr & 15), 0)
            vd_v[pl.ds(i * 16, 16)] = jnp.clip(
                c_sel - (tt - ts_sel) * STILE, 0, STILE)
            return c

        lax.fori_loop(0, NT_MAX // 16, mbody, jnp.int32(0))
        pltpu.sync_copy(tl_v, rel_hbm)
        pltpu.sync_copy(vd_v, valid_hbm)


def _route(r, lh, h, p, n):
    f = pl.kernel(
        _route_body,
        out_type=(
            jax.ShapeDtypeStruct((NSLOT,), jnp.int32),
            jax.ShapeDtypeStruct((NSLOT,), jnp.int32),
            jax.ShapeDtypeStruct((NSLOT,), jnp.int32),
            jax.ShapeDtypeStruct((NT_MAX,), jnp.int32),
            jax.ShapeDtypeStruct((NT_MAX,), jnp.int32),
        ),
        mesh=_mesh(),
        scratch_types=[
            pltpu.VMEM((SAMP_PER_W,), jnp.int32),
            pltpu.VMEM((NW * NREL,), jnp.int32),
            pltpu.VMEM((SAMP_PER_W,), jnp.int32),
            pltpu.VMEM((SAMP_PER_W,), jnp.int32),
            pltpu.VMEM((SAMP_PER_W,), jnp.int32),
            pltpu.VMEM((NREL,), jnp.int32),
            pltpu.VMEM((SAMP_PER_W // CHUNK, CHUNK), jnp.int32),
            pltpu.VMEM((NT_MAX,), jnp.int32),
            pltpu.VMEM((NT_MAX,), jnp.int32),
            pltpu.SemaphoreType.DMA,
        ],
    )
    return f(r, lh, h, p, n)


NCH = SLOTS_PER_W // CHUNK
NSTREAM = 3 * NCH


def _sgather_body(n_ent, tab_hbm, hs_hbm, ps_hbm, ns_hbm,
                  oh_hbm, op_hbm, on_hbm,
                  ih_v, ip_v, in_v, rows0_v, rows1_v, rows2_v, semg, semw):
    wid = _wid()
    base = wid * SLOTS_PER_W
    idx_hbms = (hs_hbm, ps_hbm, ns_hbm)
    out_hbms = (oh_hbm, op_hbm, on_hbm)
    idx_vs = (ih_v, ip_v, in_v)
    for e in range(3):
        pltpu.sync_copy(idx_hbms[e].at[pl.ds(base, SLOTS_PER_W)],
                        idx_vs[e])
    for e in range(3):
        for k in range(SLOTS_PER_W // 16):
            v = idx_vs[e][pl.ds(k * 16, 16)]
            idx_vs[e][pl.ds(k * 16, 16)] = jnp.clip(v, 0, n_ent - 1)

    for k in range(NSTREAM):
        e, c = divmod(k, NCH)
        pltpu.async_copy(
            tab_hbm.at[idx_vs[e].at[pl.ds(c * CHUNK, CHUNK)]],
            rows0_v, semg).wait()
        pltpu.sync_copy(rows0_v,
                        out_hbms[e].at[pl.ds(base + c * CHUNK, CHUNK)])


def _sgather(entity_embed, hs, ps, ns):
    n_ent = entity_embed.shape[0]
    f = pl.kernel(
        functools.partial(_sgather_body, n_ent),
        out_type=(
            jax.ShapeDtypeStruct((NSLOT, D), jnp.float32),
            jax.ShapeDtypeStruct((NSLOT, D), jnp.float32),
            jax.ShapeDtypeStruct((NSLOT, D), jnp.float32),
        ),
        mesh=_mesh(),
        scratch_types=[
            pltpu.VMEM((SLOTS_PER_W,), jnp.int32),
            pltpu.VMEM((SLOTS_PER_W,), jnp.int32),
            pltpu.VMEM((SLOTS_PER_W,), jnp.int32),
            pltpu.VMEM((CHUNK, D), jnp.float32),
            pltpu.VMEM((CHUNK, D), jnp.float32),
            pltpu.VMEM((CHUNK, D), jnp.float32),
            pltpu.SemaphoreType.DMA,
            pltpu.SemaphoreType.DMA,
        ],
    )
    return f(entity_embed, hs, ps, ns)


def _normalize(x):
    n = jnp.sqrt(jnp.sum(x * x, axis=1, keepdims=True))
    return x / jnp.maximum(n, 1e-12)


def _tcs_body(rel_s, valid_s, he_ref, pe_ref, ne_ref, rele_ref, wm_ref,
              out_ref):
    t = pl.program_id(0)
    W = wm_ref[0]
    X = jnp.concatenate([he_ref[...], pe_ref[...], ne_ref[...]], axis=0)
    proj = jnp.dot(X, W, preferred_element_type=jnp.float32)

    re_row = rele_ref[0]
    re_n = re_row / jnp.maximum(
        jnp.sqrt(jnp.sum(re_row * re_row)), 1e-12)

    mh = _normalize(proj[:STILE])
    mp = _normalize(proj[STILE:2 * STILE])
    mn = _normalize(proj[2 * STILE:])

    bse = mh + re_n
    pos = jnp.sqrt(jnp.sum((bse - mp) ** 2, axis=1, keepdims=True))
    neg = jnp.sqrt(jnp.sum((bse - mn) ** 2, axis=1, keepdims=True))
    kg = -jnp.log(1.0 / (1.0 + jnp.exp(pos - neg)) + 1e-08)
    l2 = 0.5 * (jnp.sum(mh * mh, axis=1, keepdims=True)
                + jnp.sum(mp * mp, axis=1, keepdims=True)
                + jnp.sum(mn * mn, axis=1, keepdims=True)
                + jnp.sum(re_n * re_n))

    valid = valid_s[t]
    rowmask = lax.broadcasted_iota(jnp.int32, (STILE, 1), 0) < valid
    zero = jnp.zeros((), jnp.float32)
    partial = (jnp.sum(jnp.where(rowmask, kg, zero))
               + LAM * jnp.sum(jnp.where(rowmask, l2, zero))) / B

    @pl.when(t == 0)
    def _init():
        out_ref[0, 0] = 0.0

    out_ref[0, 0] += partial


def _tcs(rel_t, valid_t, he, pe, ne, relation_embed, trans_M):
    grid_spec = pltpu.PrefetchScalarGridSpec(
        num_scalar_prefetch=2,
        grid=(NT_MAX,),
        in_specs=[
            pl.BlockSpec((STILE, D), lambda t, rs, vs: (t, 0)),
            pl.BlockSpec((STILE, D), lambda t, rs, vs: (t, 0)),
            pl.BlockSpec((STILE, D), lambda t, rs, vs: (t, 0)),
            pl.BlockSpec((1, 1, RD), lambda t, rs, vs: (rs[t], 0, 0)),
            pl.BlockSpec((1, D, RD), lambda t, rs, vs: (rs[t], 0, 0)),
        ],
        out_specs=pl.BlockSpec(memory_space=pltpu.SMEM),
    )
    return pl.pallas_call(
        _tcs_body,
        grid_spec=grid_spec,
        out_shape=jax.ShapeDtypeStruct((1, 1), jnp.float32),
        compiler_params=pltpu.CompilerParams(
            dimension_semantics=("arbitrary",),
        ),
    )(rel_t, valid_t, he, pe, ne,
      relation_embed.reshape(NREL, 1, RD), trans_M)


def kernel(h, r, pos_t, neg_t, entity_embed, relation_embed, trans_M):
    h = h.astype(jnp.int32)
    r = r.astype(jnp.int32)
    pos_t = pos_t.astype(jnp.int32)
    neg_t = neg_t.astype(jnp.int32)
    lh = _hist(r)
    hs, ps, ns, rel_t, valid_t = _route(r, lh, h, pos_t, neg_t)
    he, pe, ne = _sgather(entity_embed, hs, ps, ns)
    out = _tcs(rel_t, valid_t, he, pe, ne, relation_embed, trans_M)
    return out.reshape(())


# speedup vs baseline: 2.4935x; 2.4935x over previous
import functools

import jax
import jax.numpy as jnp
from jax import lax
from jax.experimental import pallas as pl
from jax.experimental.pallas import tpu as pltpu
from jax.experimental.pallas import tpu_sc as plsc

B = 16384
D = 128
RD = 128
NREL = 64
LAM = 1e-05

NC = 2
NS = 16
NW = NC * NS
ROWS_PER_W = B // NW
CHUNK = 128
NCHUNK = ROWS_PER_W // CHUNK

TILE = 256
NTILES = B // TILE
GROUP = 4
NGROUP = NREL // GROUP


def _sc_gather_body(table_hbm, h_hbm, p_hbm, n_hbm,
                    out_h, out_p, out_n,
                    idx_v, rows_v, sem):
    wid = lax.axis_index("s") * NC + lax.axis_index("c")
    base = wid * ROWS_PER_W
    for idx_hbm, out_hbm in ((h_hbm, out_h), (p_hbm, out_p), (n_hbm, out_n)):
        for c in range(NCHUNK):
            off = base + c * CHUNK
            pltpu.sync_copy(idx_hbm.at[pl.ds(off, CHUNK)], idx_v)
            pltpu.async_copy(table_hbm.at[idx_v], rows_v, sem).wait()
            pltpu.sync_copy(rows_v, out_hbm.at[pl.ds(off, CHUNK)])


def _sc_gather(entity_embed, h, p, n):
    mesh = plsc.VectorSubcoreMesh(core_axis_name="c", subcore_axis_name="s")
    f = pl.kernel(
        _sc_gather_body,
        out_type=(
            jax.ShapeDtypeStruct((B, D), jnp.float32),
            jax.ShapeDtypeStruct((B, D), jnp.float32),
            jax.ShapeDtypeStruct((B, D), jnp.float32),
        ),
        mesh=mesh,
        scratch_types=[
            pltpu.VMEM((CHUNK,), jnp.int32),
            pltpu.VMEM((CHUNK, D), jnp.float32),
            pltpu.SemaphoreType.DMA,
        ],
    )
    return f(entity_embed, h, p, n)


def _normalize(x):
    n = jnp.sqrt(jnp.sum(x * x, axis=1, keepdims=True))
    return x / jnp.maximum(n, 1e-12)


def _tc_body(r_ref, he_ref, pe_ref, ne_ref, rel_ref, wm_ref, out_ref):
    t = pl.program_id(0)
    r = r_ref[...]
    X = jnp.concatenate([he_ref[...], pe_ref[...], ne_ref[...]],
                        axis=0).astype(jnp.bfloat16)
    rr = jnp.concatenate([r, r, r], axis=0)
    X4 = jnp.concatenate([X] * GROUP, axis=1)
    lane_rel = lax.broadcasted_iota(
        jnp.int32, (3 * TILE, GROUP * D), 1) // D

    acc = jnp.zeros((3 * TILE, RD), jnp.float32)
    for j in range(NGROUP):
        mask = rr == (lane_rel + j * GROUP)
        Xm = jnp.where(mask, X4, jnp.bfloat16(0.0))
        acc = acc + jnp.dot(Xm, wm_ref[j],
                            preferred_element_type=jnp.float32)

    onehot = (r == lax.broadcasted_iota(jnp.int32, (TILE, NREL), 1))
    r_emb = jnp.dot(onehot.astype(jnp.float32), rel_ref[...],
                    preferred_element_type=jnp.float32)

    mh = _normalize(acc[:TILE])
    mp = _normalize(acc[TILE:2 * TILE])
    mn = _normalize(acc[2 * TILE:])
    re = _normalize(r_emb)

    base = mh + re
    pos = jnp.sqrt(jnp.sum((base - mp) ** 2, axis=1))
    neg = jnp.sqrt(jnp.sum((base - mn) ** 2, axis=1))
    kg = -jnp.log(1.0 / (1.0 + jnp.exp(pos - neg)) + 1e-08)
    l2 = 0.5 * (jnp.sum(mh * mh) + jnp.sum(re * re)
                + jnp.sum(mp * mp) + jnp.sum(mn * mn))
    partial = jnp.sum(kg) / B + LAM * l2 / B

    @pl.when(t == 0)
    def _init():
        out_ref[0, 0] = 0.0

    out_ref[0, 0] += partial


def _tc_loss(r2d, he, pe, ne, relation_embed, trans_M):
    return pl.pallas_call(
        _tc_body,
        grid=(NTILES,),
        in_specs=[
            pl.BlockSpec((TILE, 1), lambda t: (t, 0)),
            pl.BlockSpec((TILE, D), lambda t: (t, 0)),
            pl.BlockSpec((TILE, D), lambda t: (t, 0)),
            pl.BlockSpec((TILE, D), lambda t: (t, 0)),
            pl.BlockSpec((NREL, RD), lambda t: (0, 0)),
            pl.BlockSpec((NGROUP, GROUP * D, RD), lambda t: (0, 0, 0)),
        ],
        out_specs=pl.BlockSpec(memory_space=pltpu.SMEM),
        out_shape=jax.ShapeDtypeStruct((1, 1), jnp.float32),
        compiler_params=pltpu.CompilerParams(
            dimension_semantics=("arbitrary",),
        ),
    )(r2d, he, pe, ne, relation_embed, trans_M)


def kernel(h, r, pos_t, neg_t, entity_embed, relation_embed, trans_M):
    h = h.astype(jnp.int32)
    r = r.astype(jnp.int32)
    pos_t = pos_t.astype(jnp.int32)
    neg_t = neg_t.astype(jnp.int32)
    he, pe, ne = _sc_gather(entity_embed, h, pos_t, neg_t)
    wm4 = trans_M.astype(jnp.bfloat16).reshape(NGROUP, GROUP * D, RD)
    out = _tc_loss(r.reshape(B, 1), he, pe, ne, relation_embed, wm4)
    return out.reshape(())
